# baseline (speedup 1.0000x reference)
import functools

import jax
import jax.numpy as jnp
import numpy as np
from jax import lax
from jax.experimental import pallas as pl
from jax.experimental.pallas import tpu as pltpu
from jax.experimental.pallas import tpu_sc as plsc

B, F, V, D = 16384, 26, 1000000, 16
H1, H2 = 256, 128
FP = 32
NQ = 4
NC, NS = 2, 16
NW = NC * NS
BPT = B // NW
UB = 16
NU = BPT // UB
IDXR = B * 8 // 128


def _sc_gather(emb_v, w16, idx_qs, widx_qs):
    mesh = plsc.VectorSubcoreMesh(core_axis_name="c", subcore_axis_name="s")
    out1 = jax.ShapeDtypeStruct((B * 8, 16), jnp.float32)

    @functools.partial(
        pl.kernel,
        mesh=mesh,
        compiler_params=pltpu.CompilerParams(
            use_tc_tiling_on_sc=False, needs_layout_passes=False),
        out_type=(out1,) * 8,
        scratch_types=[
            pltpu.VMEM((8, 128), jnp.int32),
            pltpu.VMEM((1024, 16), jnp.float32),
            pltpu.SemaphoreType.DMA,
            pltpu.SemaphoreType.DMA,
        ],
    )
    def k(emb_hbm, w_hbm, i0, i1, i2, i3, wi0, wi1, wi2, wi3,
          e0, e1, e2, e3, w0, w1, w2, w3, idx_v, stage, sem_e, sem_w):
        wid = lax.axis_index("s") * NC + lax.axis_index("c")
        idx_refs = (i0, i1, i2, i3, wi0, wi1, wi2, wi3)
        out_refs = (e0, e1, e2, e3, w0, w1, w2, w3)

        @pl.loop(0, NU)
        def _(u):
            row = wid * NU + u
            for q in range(8):
                pltpu.sync_copy(idx_refs[q].at[pl.ds(row, 1)],
                                idx_v.at[pl.ds(q, 1)])
            cps = []
            for q in range(8):
                sem = sem_e if q < 4 else sem_w
                src = emb_hbm if q < 4 else w_hbm
                cps.append(pltpu.async_copy(
                    src.at[idx_v.at[q]],
                    stage.at[pl.ds(q * 128, 128)], sem))
            for cp in cps:
                cp.wait()
            base = row * 128
            for q in range(8):
                pltpu.sync_copy(
                    stage.at[pl.ds(q * 128, 128)],
                    out_refs[q].at[pl.ds(base, 128)])

    return k(emb_v, w16, *idx_qs, *widx_qs)


BBLK = 1024


def _fm_mlp_body(e0, e1, e2, e3, w0, w1, w2, w3, vals_ref, lo_ref,
                 W1q0, W1q1, W1q2, W1q3, b1_ref, W2_ref, b2_ref,
                 W3_ref, b3f_ref, R0, R1, R2, R3, S_ref, LM_ref, out_ref):
    vals = vals_ref[...]
    lo = lo_ref[...]
    S = S_ref[...]
    LM = LM_ref[...]
    e_refs = (e0, e1, e2, e3)
    w_refs = (w0, w1, w2, w3)
    R_refs = (R0, R1, R2, R3)
    W1_refs = (W1q0, W1q1, W1q2, W1q3)
    first = jnp.zeros((BBLK,), jnp.float32)
    sum_e = jnp.zeros((BBLK, D), jnp.float32)
    sum_sq = jnp.zeros((BBLK, D), jnp.float32)
    h = b1_ref[...] * jnp.ones((BBLK, 1), jnp.float32)
    for q in range(NQ):
        R = R_refs[q][...]
        vr = jnp.dot(vals, R, preferred_element_type=jnp.float32)
        lr = jnp.dot(lo, R, preferred_element_type=jnp.float32)
        ev = e_refs[q][...] * vr
        sum_e = sum_e + jnp.dot(ev, S, preferred_element_type=jnp.float32)
        sum_sq = sum_sq + jnp.dot(ev * ev, S,
                                  preferred_element_type=jnp.float32)
        h = h + jnp.dot(ev, W1_refs[q][...],
                        preferred_element_type=jnp.float32)
        msk = jnp.where(lr == LM, 1.0, 0.0)
        first = first + jnp.sum(w_refs[q][...] * vr * msk, axis=1)
    second = 0.5 * jnp.sum(sum_e * sum_e - sum_sq, axis=1)
    h = jnp.maximum(h, 0.0)
    h = jnp.maximum(jnp.dot(h, W2_ref[...], preferred_element_type=jnp.float32)
                    + b2_ref[...], 0.0)
    deep = jnp.dot(h, W3_ref[...], preferred_element_type=jnp.float32)[:, 0]
    logits = first + second + deep + b3f_ref[0, 0]
    out_ref[...] = 1.0 / (1.0 + jnp.exp(-logits))


def _fm_mlp(e_qs, w_qs, vals, lo, W1qs, b1, W2, b2, W3, b3f, Rqs, S128, LM):
    grid = (B // BBLK,)
    blk = lambda *s: [pl.BlockSpec(s, lambda i: (i, 0))]
    cst = lambda *s: [pl.BlockSpec(s, lambda i: (0, 0))]
    in_specs = (blk(BBLK, 128) * 4 + blk(BBLK, 128) * 4
                + blk(BBLK, F) + blk(BBLK, F)
                + cst(128, H1) * 4 + cst(1, H1) + cst(H1, H2) + cst(1, H2)
                + cst(H2, 1) + cst(1, 1) + cst(F, 128) * 4
                + cst(128, D) + cst(1, 128))
    return pl.pallas_call(
        _fm_mlp_body,
        grid=grid,
        in_specs=in_specs,
        out_specs=pl.BlockSpec((BBLK,), lambda i: (i,)),
        out_shape=jax.ShapeDtypeStruct((B,), jnp.float32),
    )(*e_qs, *w_qs, vals, lo, *W1qs, b1, W2, b2, W3, b3f, *Rqs, S128, LM)


_Rq_np = []
for _q in range(NQ):
    _r = np.zeros((F, 128), dtype=np.float32)
    for _j in range(128):
        _f = 8 * _q + _j // 16
        if _f < F:
            _r[_f, _j] = 1.0
    _Rq_np.append(_r)
_S128_np = np.zeros((128, D), dtype=np.float32)
for _j in range(128):
    _S128_np[_j, _j % 16] = 1.0
_LM_np = (np.arange(128, dtype=np.float32) % 16).reshape(1, 128)


def kernel(feat_ids, feat_vals, w_first, emb_v, W1, b1, W2, b2, W3, b3, bias):
    idsp = jnp.concatenate(
        [feat_ids, jnp.zeros((B, FP - F), jnp.int32)], axis=1)
    hi = idsp >> 4
    idx_qs = [idsp[:, 8 * q:8 * (q + 1)].reshape(IDXR, 128) for q in range(NQ)]
    widx_qs = [hi[:, 8 * q:8 * (q + 1)].reshape(IDXR, 128) for q in range(NQ)]
    w16 = w_first.reshape(V // 16, 16)
    outs = [o.reshape(B, 128) for o in _sc_gather(emb_v, w16, idx_qs, widx_qs)]
    e_qs, w_qs = outs[:4], outs[4:]
    lo = (feat_ids & 15).astype(jnp.float32)
    W1qs = [W1[128 * q:128 * (q + 1)] for q in range(3)]
    W1qs.append(jnp.concatenate(
        [W1[384:416], jnp.zeros((128 - 32, H1), jnp.float32)], axis=0))
    b3f = (b3 + bias).reshape(1, 1)
    Rqs = [jnp.asarray(r) for r in _Rq_np]
    return _fm_mlp(e_qs, w_qs, feat_vals, lo, W1qs, b1.reshape(1, H1),
                   W2, b2.reshape(1, H2), W3, b3f, Rqs,
                   jnp.asarray(_S128_np), jnp.asarray(_LM_np))


# speedup vs baseline: 1.0011x; 1.0011x over previous
import functools

import jax
import jax.numpy as jnp
import numpy as np
from jax import lax
from jax.experimental import pallas as pl
from jax.experimental.pallas import tpu as pltpu
from jax.experimental.pallas import tpu_sc as plsc

B, F, V, D = 16384, 26, 1000000, 16
H1, H2 = 256, 128
FP = 32
NQ = 4
NC, NS = 2, 16
NW = NC * NS
BPT = B // NW
UB = 16
NU = BPT // UB
IDXR = B * 8 // 128


def _sc_gather(emb_v, w16, idx_qs, widx_qs):
    mesh = plsc.VectorSubcoreMesh(core_axis_name="c", subcore_axis_name="s")
    out1 = jax.ShapeDtypeStruct((B, 128), jnp.float32)
    CB = 32
    CR = CB * 8 // 128
    NCH = BPT // CB

    @functools.partial(
        pl.kernel,
        mesh=mesh,
        compiler_params=pltpu.CompilerParams(
            use_tc_tiling_on_sc=False, needs_layout_passes=False),
        out_type=(out1,) * 8,
        scratch_types=[
            pltpu.VMEM((8 * CR, 128), jnp.int32),
            pltpu.VMEM((8 * CR * 128, 16), jnp.float32),
            pltpu.VMEM((8 * CB, 128), jnp.float32),
            pltpu.SemaphoreType.DMA,
            pltpu.SemaphoreType.DMA,
        ],
    )
    def k(emb_hbm, w_hbm, i0, i1, i2, i3, wi0, wi1, wi2, wi3,
          e0, e1, e2, e3, w0, w1, w2, w3, idx_v, stage, stage2,
          sem_e, sem_w):
        wid = lax.axis_index("s") * NC + lax.axis_index("c")
        idx_refs = (i0, i1, i2, i3, wi0, wi1, wi2, wi3)
        out_refs = (e0, e1, e2, e3, w0, w1, w2, w3)

        @pl.loop(0, NCH)
        def _(c):
            row = (wid * NCH + c) * CR
            for q in range(8):
                pltpu.sync_copy(idx_refs[q].at[pl.ds(row, CR)],
                                idx_v.at[pl.ds(q * CR, CR)])
            cps = []
            for q in range(8):
                sem = sem_e if q < 4 else sem_w
                src = emb_hbm if q < 4 else w_hbm
                for j in range(CR):
                    cps.append(pltpu.async_copy(
                        src.at[idx_v.at[q * CR + j]],
                        stage.at[pl.ds((q * CR + j) * 128, 128)], sem))
            for cp in cps:
                cp.wait()

            @pl.loop(0, CB)
            def _(b):
                for q in range(8):
                    for s in range(8):
                        r = (q * CB + b) * 8 + s
                        stage2[q * CB + b, pl.ds(s * 16, 16)] = (
                            stage[r, pl.ds(0, 16)])

            b0 = (wid * NCH + c) * CB
            for q in range(8):
                pltpu.sync_copy(stage2.at[pl.ds(q * CB, CB)],
                                out_refs[q].at[pl.ds(b0, CB)])

    return k(emb_v, w16, *idx_qs, *widx_qs)


BBLK = 1024


def _fm_mlp_body(e0, e1, e2, e3, w0, w1, w2, w3, vals_ref, lo_ref,
                 W1q0, W1q1, W1q2, W1q3, b1_ref, W2_ref, b2_ref,
                 W3_ref, b3f_ref, R0, R1, R2, R3, S_ref, LM_ref, out_ref):
    vals = vals_ref[...]
    lo = lo_ref[...]
    S = S_ref[...]
    LM = LM_ref[...]
    e_refs = (e0, e1, e2, e3)
    w_refs = (w0, w1, w2, w3)
    R_refs = (R0, R1, R2, R3)
    W1_refs = (W1q0, W1q1, W1q2, W1q3)
    first = jnp.zeros((BBLK,), jnp.float32)
    sum_e = jnp.zeros((BBLK, D), jnp.float32)
    sum_sq = jnp.zeros((BBLK, D), jnp.float32)
    h = b1_ref[...] * jnp.ones((BBLK, 1), jnp.float32)
    for q in range(NQ):
        R = R_refs[q][...]
        vr = jnp.dot(vals, R, preferred_element_type=jnp.float32)
        lr = jnp.dot(lo, R, preferred_element_type=jnp.float32)
        ev = e_refs[q][...] * vr
        sum_e = sum_e + jnp.dot(ev, S, preferred_element_type=jnp.float32)
        sum_sq = sum_sq + jnp.dot(ev * ev, S,
                                  preferred_element_type=jnp.float32)
        h = h + jnp.dot(ev, W1_refs[q][...],
                        preferred_element_type=jnp.float32)
        msk = jnp.where(lr == LM, 1.0, 0.0)
        first = first + jnp.sum(w_refs[q][...] * vr * msk, axis=1)
    second = 0.5 * jnp.sum(sum_e * sum_e - sum_sq, axis=1)
    h = jnp.maximum(h, 0.0)
    h = jnp.maximum(jnp.dot(h, W2_ref[...], preferred_element_type=jnp.float32)
                    + b2_ref[...], 0.0)
    deep = jnp.dot(h, W3_ref[...], preferred_element_type=jnp.float32)[:, 0]
    logits = first + second + deep + b3f_ref[0, 0]
    out_ref[...] = 1.0 / (1.0 + jnp.exp(-logits))


def _fm_mlp(e_qs, w_qs, vals, lo, W1qs, b1, W2, b2, W3, b3f, Rqs, S128, LM):
    grid = (B // BBLK,)
    blk = lambda *s: [pl.BlockSpec(s, lambda i: (i, 0))]
    cst = lambda *s: [pl.BlockSpec(s, lambda i: (0, 0))]
    in_specs = (blk(BBLK, 128) * 4 + blk(BBLK, 128) * 4
                + blk(BBLK, F) + blk(BBLK, F)
                + cst(128, H1) * 4 + cst(1, H1) + cst(H1, H2) + cst(1, H2)
                + cst(H2, 1) + cst(1, 1) + cst(F, 128) * 4
                + cst(128, D) + cst(1, 128))
    return pl.pallas_call(
        _fm_mlp_body,
        grid=grid,
        in_specs=in_specs,
        out_specs=pl.BlockSpec((BBLK,), lambda i: (i,)),
        out_shape=jax.ShapeDtypeStruct((B,), jnp.float32),
    )(*e_qs, *w_qs, vals, lo, *W1qs, b1, W2, b2, W3, b3f, *Rqs, S128, LM)


_Rq_np = []
for _q in range(NQ):
    _r = np.zeros((F, 128), dtype=np.float32)
    for _j in range(128):
        _f = 8 * _q + _j // 16
        if _f < F:
            _r[_f, _j] = 1.0
    _Rq_np.append(_r)
_S128_np = np.zeros((128, D), dtype=np.float32)
for _j in range(128):
    _S128_np[_j, _j % 16] = 1.0
_LM_np = (np.arange(128, dtype=np.float32) % 16).reshape(1, 128)


def kernel(feat_ids, feat_vals, w_first, emb_v, W1, b1, W2, b2, W3, b3, bias):
    idsp = jnp.concatenate(
        [feat_ids, jnp.zeros((B, FP - F), jnp.int32)], axis=1)
    hi = idsp >> 4
    idx_qs = [idsp[:, 8 * q:8 * (q + 1)].reshape(IDXR, 128) for q in range(NQ)]
    widx_qs = [hi[:, 8 * q:8 * (q + 1)].reshape(IDXR, 128) for q in range(NQ)]
    w16 = w_first.reshape(V // 16, 16)
    outs = _sc_gather(emb_v, w16, idx_qs, widx_qs)
    e_qs, w_qs = outs[:4], outs[4:]
    lo = (feat_ids & 15).astype(jnp.float32)
    W1qs = [W1[128 * q:128 * (q + 1)] for q in range(3)]
    W1qs.append(jnp.concatenate(
        [W1[384:416], jnp.zeros((128 - 32, H1), jnp.float32)], axis=0))
    b3f = (b3 + bias).reshape(1, 1)
    Rqs = [jnp.asarray(r) for r in _Rq_np]
    return _fm_mlp(e_qs, w_qs, feat_vals, lo, W1qs, b1.reshape(1, H1),
                   W2, b2.reshape(1, H2), W3, b3f, Rqs,
                   jnp.asarray(_S128_np), jnp.asarray(_LM_np))


# speedup vs baseline: 1.0211x; 1.0200x over previous
import functools

import jax
import jax.numpy as jnp
import numpy as np
from jax import lax
from jax.experimental import pallas as pl
from jax.experimental.pallas import tpu as pltpu
from jax.experimental.pallas import tpu_sc as plsc

B, F, V, D = 16384, 26, 1000000, 16
H1, H2 = 256, 128
FP = 32
NQ = 4
NC, NS = 2, 16
NW = NC * NS
BPT = B // NW
UB = 16
NU = BPT // UB
IDXR = B * 8 // 128


def _sc_gather(emb_v, w_first, idx_qs, widx):
    mesh = plsc.VectorSubcoreMesh(core_axis_name="c", subcore_axis_name="s")
    oute = jax.ShapeDtypeStruct((B * 8, 16), jnp.float32)
    outw = jax.ShapeDtypeStruct((B * FP,), jnp.float32)
    CB = 32
    CR = CB * 8 // 128
    WR = CB * FP // 128
    NCH = BPT // CB

    @functools.partial(
        pl.kernel,
        mesh=mesh,
        compiler_params=pltpu.CompilerParams(
            use_tc_tiling_on_sc=False, needs_layout_passes=False),
        out_type=(oute,) * 4 + (outw,),
        scratch_types=[
            pltpu.VMEM((4 * CR, 128), jnp.int32),
            pltpu.VMEM((WR, 128), jnp.int32),
            pltpu.VMEM((4 * CR * 128, 16), jnp.float32),
            pltpu.VMEM((CB * FP,), jnp.float32),
            pltpu.SemaphoreType.DMA,
            pltpu.SemaphoreType.DMA,
        ],
    )
    def k(emb_hbm, w_hbm, i0, i1, i2, i3, widx_hbm,
          e0, e1, e2, e3, wf, idx_v, widx_v, stage, wstage, sem_e, sem_w):
        wid = lax.axis_index("s") * NC + lax.axis_index("c")
        idx_refs = (i0, i1, i2, i3)
        out_refs = (e0, e1, e2, e3)

        @pl.loop(0, NCH)
        def _(c):
            row = (wid * NCH + c) * CR
            for q in range(4):
                pltpu.sync_copy(idx_refs[q].at[pl.ds(row, CR)],
                                idx_v.at[pl.ds(q * CR, CR)])
            wrow = (wid * NCH + c) * WR
            pltpu.sync_copy(widx_hbm.at[pl.ds(wrow, WR)], widx_v)
            cps = []
            for q in range(4):
                for j in range(CR):
                    cps.append(pltpu.async_copy(
                        emb_hbm.at[idx_v.at[q * CR + j]],
                        stage.at[pl.ds((q * CR + j) * 128, 128)], sem_e))
            for j in range(WR):
                cps.append(pltpu.async_copy(
                    w_hbm.at[widx_v.at[j]],
                    wstage.at[pl.ds(j * 128, 128)], sem_w))
            for cp in cps:
                cp.wait()
            base = (wid * NCH + c) * CB * 8
            for q in range(4):
                pltpu.sync_copy(stage.at[pl.ds(q * CR * 128, CR * 128)],
                                out_refs[q].at[pl.ds(base, CR * 128)])
            pltpu.sync_copy(wstage, wf.at[pl.ds((wid * NCH + c) * CB * FP,
                                               CB * FP)])

    return k(emb_v, w_first, *idx_qs, widx)


BBLK = 1024


def _fm_mlp_body(e0, e1, e2, e3, wf_ref, valp_ref, vals_ref,
                 W1q0, W1q1, W1q2, W1q3, b1_ref, W2_ref, b2_ref,
                 W3_ref, b3f_ref, R0, R1, R2, R3, S_ref, G32_ref, A4_ref,
                 M4_ref, out_ref):
    vals = vals_ref[...]
    S = S_ref[...]
    e_refs = (e0, e1, e2, e3)
    R_refs = (R0, R1, R2, R3)
    W1_refs = (W1q0, W1q1, W1q2, W1q3)
    sum_e = jnp.zeros((BBLK, D), jnp.float32)
    sum_sq = jnp.zeros((BBLK, D), jnp.float32)
    h = b1_ref[...] * jnp.ones((BBLK, 1), jnp.float32)
    for q in range(NQ):
        R = R_refs[q][...]
        vr = jnp.dot(vals, R, preferred_element_type=jnp.float32)
        ev = e_refs[q][...] * vr
        sum_e = sum_e + jnp.dot(ev, S, preferred_element_type=jnp.float32)
        sum_sq = sum_sq + jnp.dot(ev * ev, S,
                                  preferred_element_type=jnp.float32)
        h = h + jnp.dot(ev, W1_refs[q][...],
                        preferred_element_type=jnp.float32)
    X = wf_ref[...] * valp_ref[...]
    F4 = jnp.dot(X, G32_ref[...], preferred_element_type=jnp.float32)
    Z = jnp.dot(A4_ref[...], F4, preferred_element_type=jnp.float32)
    first = jnp.sum(Z * M4_ref[...], axis=1)
    second = 0.5 * jnp.sum(sum_e * sum_e - sum_sq, axis=1)
    h = jnp.maximum(h, 0.0)
    h = jnp.maximum(jnp.dot(h, W2_ref[...], preferred_element_type=jnp.float32)
                    + b2_ref[...], 0.0)
    deep = jnp.dot(h, W3_ref[...], preferred_element_type=jnp.float32)[:, 0]
    logits = first + second + deep + b3f_ref[0, 0]
    out_ref[...] = 1.0 / (1.0 + jnp.exp(-logits))


def _fm_mlp(e_qs, wfp, valp, vals, W1qs, b1, W2, b2, W3, b3f, Rqs, S128,
            G32, A4, M4):
    grid = (B // BBLK,)
    blk = lambda *s: [pl.BlockSpec(s, lambda i: (i, 0))]
    cst = lambda *s: [pl.BlockSpec(s, lambda i: (0, 0))]
    in_specs = (blk(BBLK, 128) * 4
                + blk(BBLK // 4, 128) + blk(BBLK // 4, 128)
                + blk(BBLK, F)
                + cst(128, H1) * 4 + cst(1, H1) + cst(H1, H2) + cst(1, H2)
                + cst(H2, 1) + cst(1, 1) + cst(F, 128) * 4
                + cst(128, D) + cst(128, 4) + cst(BBLK, BBLK // 4)
                + cst(BBLK, 4))
    return pl.pallas_call(
        _fm_mlp_body,
        grid=grid,
        in_specs=in_specs,
        out_specs=pl.BlockSpec((BBLK,), lambda i: (i,)),
        out_shape=jax.ShapeDtypeStruct((B,), jnp.float32),
    )(*e_qs, wfp, valp, vals, *W1qs, b1, W2, b2, W3, b3f, *Rqs, S128,
      G32, A4, M4)


_Rq_np = []
for _q in range(NQ):
    _r = np.zeros((F, 128), dtype=np.float32)
    for _j in range(128):
        _f = 8 * _q + _j // 16
        if _f < F:
            _r[_f, _j] = 1.0
    _Rq_np.append(_r)
_S128_np = np.zeros((128, D), dtype=np.float32)
for _j in range(128):
    _S128_np[_j, _j % 16] = 1.0
_G32_np = np.zeros((128, 4), dtype=np.float32)
for _j in range(128):
    _G32_np[_j, _j // 32] = 1.0
_BB = 1024
_A4_np = np.zeros((_BB, _BB // 4), dtype=np.float32)
_A4_np[np.arange(_BB), np.arange(_BB) // 4] = 1.0
_M4_np = np.zeros((_BB, 4), dtype=np.float32)
_M4_np[np.arange(_BB), np.arange(_BB) % 4] = 1.0


def kernel(feat_ids, feat_vals, w_first, emb_v, W1, b1, W2, b2, W3, b3, bias):
    idsp = jnp.concatenate(
        [feat_ids, jnp.zeros((B, FP - F), jnp.int32)], axis=1)
    idx_qs = [idsp[:, 8 * q:8 * (q + 1)].reshape(IDXR, 128) for q in range(NQ)]
    widx = idsp.reshape(B * FP // 128, 128)
    outs = _sc_gather(emb_v, w_first, idx_qs, widx)
    e_qs = [o.reshape(B, 128) for o in outs[:4]]
    wfp = outs[4].reshape(B * FP // 128, 128)
    valp = jnp.concatenate(
        [feat_vals, jnp.zeros((B, FP - F), jnp.float32)],
        axis=1).reshape(B * FP // 128, 128)
    W1qs = [W1[128 * q:128 * (q + 1)] for q in range(3)]
    W1qs.append(jnp.concatenate(
        [W1[384:416], jnp.zeros((128 - 32, H1), jnp.float32)], axis=0))
    b3f = (b3 + bias).reshape(1, 1)
    Rqs = [jnp.asarray(r) for r in _Rq_np]
    return _fm_mlp(e_qs, wfp, valp, feat_vals, W1qs, b1.reshape(1, H1),
                   W2, b2.reshape(1, H2), W3, b3f, Rqs,
                   jnp.asarray(_S128_np), jnp.asarray(_G32_np),
                   jnp.asarray(_A4_np), jnp.asarray(_M4_np))


# speedup vs baseline: 1.7813x; 1.7445x over previous
import functools

import jax
import jax.numpy as jnp
import numpy as np
from jax import lax
from jax.experimental import pallas as pl
from jax.experimental.pallas import tpu as pltpu
from jax.experimental.pallas import tpu_sc as plsc

B, F, V, D = 16384, 26, 1000000, 16
H1, H2 = 256, 128
N = B * F
NC, NS = 2, 16
NW = NC * NS
IDXW = 128
CHUNK_ROWS = 8
CHUNK = CHUNK_ROWS * IDXW
N_IDX_ROWS = N // IDXW
ROWS_PER_W = N_IDX_ROWS // NW
N_CHUNKS = ROWS_PER_W // CHUNK_ROWS


def _sc_gather(emb_v, w_first, idx2):
    mesh = plsc.VectorSubcoreMesh(core_axis_name="c", subcore_axis_name="s")

    @functools.partial(
        pl.kernel,
        mesh=mesh,
        compiler_params=pltpu.CompilerParams(
            use_tc_tiling_on_sc=False, needs_layout_passes=False),
        out_type=(
            jax.ShapeDtypeStruct((N, D), jnp.float32),
            jax.ShapeDtypeStruct((N,), jnp.float32),
        ),
        scratch_types=[
            pltpu.VMEM((CHUNK_ROWS, IDXW), jnp.int32),
            pltpu.VMEM((CHUNK, D), jnp.float32),
            pltpu.VMEM((CHUNK,), jnp.float32),
            pltpu.SemaphoreType.DMA,
            pltpu.SemaphoreType.DMA,
        ],
    )
    def k(emb_hbm, w_hbm, idx_hbm, e_out, w_out,
          idx_v, rows_v, wv_v, sem_e, sem_w):
        wid = lax.axis_index("s") * NC + lax.axis_index("c")
        row_base = wid * ROWS_PER_W

        @pl.loop(0, N_CHUNKS)
        def _(c):
            r0 = row_base + c * CHUNK_ROWS
            pltpu.sync_copy(idx_hbm.at[pl.ds(r0, CHUNK_ROWS)], idx_v)
            cps = []
            for j in range(CHUNK_ROWS):
                cps.append(pltpu.async_copy(
                    emb_hbm.at[idx_v.at[j]],
                    rows_v.at[pl.ds(j * IDXW, IDXW)], sem_e))
                cps.append(pltpu.async_copy(
                    w_hbm.at[idx_v.at[j]],
                    wv_v.at[pl.ds(j * IDXW, IDXW)], sem_w))
            for cp in cps:
                cp.wait()
            base = r0 * IDXW
            pltpu.sync_copy(rows_v, e_out.at[pl.ds(base, CHUNK)])
            pltpu.sync_copy(wv_v, w_out.at[pl.ds(base, CHUNK)])

    return k(emb_v, w_first, idx2)


BBLK = 1024


def _fm_mlp_body(e_ref, vals_ref, wf_ref, W1_ref, b1_ref, W2_ref, b2_ref,
                 W3_ref, b3f_ref, R_ref, S_ref, out_ref):
    vals = vals_ref[...]
    vrep = jnp.dot(vals, R_ref[...], preferred_element_type=jnp.float32)
    ev = e_ref[...] * vrep
    first = jnp.sum(wf_ref[...] * vals, axis=1)
    S = S_ref[...]
    sum_e = jnp.dot(ev, S, preferred_element_type=jnp.float32)
    sum_sq = jnp.dot(ev * ev, S, preferred_element_type=jnp.float32)
    second = 0.5 * jnp.sum(sum_e * sum_e - sum_sq, axis=1)
    h = jnp.maximum(jnp.dot(ev, W1_ref[...], preferred_element_type=jnp.float32)
                    + b1_ref[...], 0.0)
    h = jnp.maximum(jnp.dot(h, W2_ref[...], preferred_element_type=jnp.float32)
                    + b2_ref[...], 0.0)
    deep = jnp.dot(h, W3_ref[...], preferred_element_type=jnp.float32)[:, 0]
    logits = first + second + deep + b3f_ref[0, 0]
    out_ref[...] = 1.0 / (1.0 + jnp.exp(-logits))


def _fm_mlp(e2, vals, wf, W1, b1, W2, b2, W3, b3f, R, S):
    grid = (B // BBLK,)
    return pl.pallas_call(
        _fm_mlp_body,
        grid=grid,
        in_specs=[
            pl.BlockSpec((BBLK, F * D), lambda i: (i, 0)),
            pl.BlockSpec((BBLK, F), lambda i: (i, 0)),
            pl.BlockSpec((BBLK, F), lambda i: (i, 0)),
            pl.BlockSpec((F * D, H1), lambda i: (0, 0)),
            pl.BlockSpec((1, H1), lambda i: (0, 0)),
            pl.BlockSpec((H1, H2), lambda i: (0, 0)),
            pl.BlockSpec((1, H2), lambda i: (0, 0)),
            pl.BlockSpec((H2, 1), lambda i: (0, 0)),
            pl.BlockSpec((1, 1), lambda i: (0, 0)),
            pl.BlockSpec((F, F * D), lambda i: (0, 0)),
            pl.BlockSpec((F * D, D), lambda i: (0, 0)),
        ],
        out_specs=pl.BlockSpec((BBLK,), lambda i: (i,)),
        out_shape=jax.ShapeDtypeStruct((B,), jnp.float32),
    )(e2, vals, wf, W1, b1, W2, b2, W3, b3f, R, S)


_R_np = np.zeros((F, F * D), dtype=np.float32)
for _f in range(F):
    _R_np[_f, _f * D:(_f + 1) * D] = 1.0
_S_np = np.zeros((F * D, D), dtype=np.float32)
for _f in range(F):
    _S_np[np.arange(_f * D, (_f + 1) * D), np.arange(D)] = 1.0


def kernel(feat_ids, feat_vals, w_first, emb_v, W1, b1, W2, b2, W3, b3, bias):
    idx2 = feat_ids.reshape(N_IDX_ROWS, IDXW)
    e_raw, wf_flat = _sc_gather(emb_v, w_first, idx2)
    e2 = e_raw.reshape(B, F * D)
    wf = wf_flat.reshape(B, F)
    b3f = (b3 + bias).reshape(1, 1)
    R = jnp.asarray(_R_np)
    S = jnp.asarray(_S_np)
    return _fm_mlp(e2, feat_vals, wf, W1, b1.reshape(1, H1), W2,
                   b2.reshape(1, H2), W3, b3f, R, S)
